# baseline (speedup 1.0000x reference)
import functools

import jax
import jax.numpy as jnp
from jax import lax
from jax.experimental import pallas as pl
from jax.experimental.pallas import tpu as pltpu
from jax.experimental.pallas import tpu_sc as plsc

N = 10000
IN_DIM = 15
HID = 32
ENC = 256
SPACE = 2
K = 16
OUT_ID = 6
OUT_P4 = 4
ELEM_OFF = 3

TR = 1000
TM = 2000

_INTERPRET = False


def _leaky(v):
    return jnp.where(v >= 0, v, 0.01 * v)


def _sh_kernel(x_ref, Ws_ref, bs_ref, Wh_ref, bh_ref, s_ref, sq_ref, h_ref):
    x = x_ref[...]
    s = jnp.dot(x, Ws_ref[...], preferred_element_type=jnp.float32) + bs_ref[...]
    s_ref[...] = s
    sq_ref[...] = jnp.sum(s * s, axis=1, keepdims=True)
    h_ref[...] = jnp.dot(x, Wh_ref[...], preferred_element_type=jnp.float32) + bh_ref[...]


def _knn_kernel(sq_col_ref, s_tile_ref, sT_ref, sq_row_ref,
                nbr_ref, wgt_ref, d2_ref):
    i = pl.program_id(0)
    p = jnp.dot(s_tile_ref[...], sT_ref[...], preferred_element_type=jnp.float32)
    d2 = (sq_col_ref[...] + sq_row_ref[...]) - 2.0 * p
    rows = jax.lax.broadcasted_iota(jnp.int32, (TR, N), 0) + i * TR
    cols = jax.lax.broadcasted_iota(jnp.int32, (TR, N), 1)
    d2_ref[...] = jnp.where(cols == rows, jnp.inf, d2)

    def body(k, carry):
        nbr_acc, wgt_acc = carry
        d2c = d2_ref[...]
        m = jnp.min(d2c, axis=1, keepdims=True)
        idx = jnp.argmin(d2c, axis=1, keepdims=True).astype(jnp.int32)
        d2_ref[...] = jnp.where(cols == idx, jnp.inf, d2c)
        lane = jax.lax.broadcasted_iota(jnp.int32, (TR, K), 1)
        nbr_acc = jnp.where(lane == k, idx, nbr_acc)
        wgt_acc = jnp.where(lane == k, jnp.exp(-10.0 * m), wgt_acc)
        return nbr_acc, wgt_acc

    nbr, wgt = jax.lax.fori_loop(
        0, K,
        body,
        (jnp.zeros((TR, K), jnp.int32),
         jnp.zeros((TR, K), jnp.float32)),
    )
    nbr_ref[...] = nbr
    wgt_ref[...] = wgt


N_P = 10240
HPAD = 128
SC_RC = 64
SC_CH = 5
SC_G = SC_RC * K // 128
SC_G2 = SC_G // 2


def _sc_agg_body(h_hbm, nbr2d_hbm, wgt_hbm, mean_hbm, max_hbm,
                 idx_v, rows_v, wv_all, mean_v, max_v, sem):
    wid = lax.axis_index("s") * 2 + lax.axis_index("c")
    k_bcast_idx = [jnp.full((16,), kk, jnp.int32)[:, None] for kk in range(K)]
    dnums = lax.GatherDimensionNumbers(offset_dims=(), collapsed_slice_dims=(0,),
                                       start_index_map=(0,))

    for c in range(SC_CH):
        r0 = pl.multiple_of(wid * (SC_RC * SC_CH) + c * SC_RC, SC_RC)
        pltpu.sync_copy(nbr2d_hbm.at[pl.ds(pl.multiple_of(r0 // 8, 8), SC_G)], idx_v)
        pltpu.sync_copy(wgt_hbm.at[pl.ds(r0, SC_RC)], wv_all)
        for half in range(2):
            copies = [pltpu.async_copy(h_hbm.at[idx_v.at[half * SC_G2 + j]],
                                       rows_v.at[pl.ds(j * 128, 128)], sem)
                      for j in range(SC_G2)]
            for cp in copies:
                cp.wait()

            def row_body(r, _):
                wv = wv_all[half * (SC_RC // 2) + r, :]
                acc0 = jnp.zeros((16,), jnp.float32)
                acc1 = jnp.zeros((16,), jnp.float32)
                mx0 = jnp.full((16,), -jnp.inf, jnp.float32)
                mx1 = jnp.full((16,), -jnp.inf, jnp.float32)
                for kk in range(K):
                    wb = lax.gather(wv, k_bcast_idx[kk], dnums, (1,),
                                    mode=lax.GatherScatterMode.PROMISE_IN_BOUNDS)
                    g0 = rows_v[r * K + kk, 0:16]
                    g1 = rows_v[r * K + kk, 16:32]
                    m0 = wb * g0
                    m1 = wb * g1
                    acc0 = acc0 + m0
                    acc1 = acc1 + m1
                    mx0 = jnp.maximum(mx0, m0)
                    mx1 = jnp.maximum(mx1, m1)
                ro = half * (SC_RC // 2) + r
                mean_v[ro, 0:16] = acc0 * (1.0 / K)
                mean_v[ro, 16:32] = acc1 * (1.0 / K)
                max_v[ro, 0:16] = mx0
                max_v[ro, 16:32] = mx1
                return 0

            lax.fori_loop(0, SC_RC // 2, row_body, 0)
        pltpu.sync_copy(mean_v, mean_hbm.at[pl.ds(r0, SC_RC)])
        pltpu.sync_copy(max_v, max_hbm.at[pl.ds(r0, SC_RC)])


def _sc_aggregate(h, nbr, wgt):
    pad = N_P - N
    nbr2d = jnp.pad(nbr, ((0, pad), (0, 0))).reshape(N_P * K // 128, 128)
    wgt_p = jnp.pad(wgt, ((0, pad), (0, 0)))
    f32 = jnp.float32
    mesh = plsc.VectorSubcoreMesh(core_axis_name="c", subcore_axis_name="s")
    run = pl.kernel(
        _sc_agg_body,
        mesh=mesh,
        out_type=(jax.ShapeDtypeStruct((N_P, HID), f32),
                  jax.ShapeDtypeStruct((N_P, HID), f32)),
        scratch_types=[
            pltpu.VMEM((SC_G, 128), jnp.int32),
            pltpu.VMEM((SC_RC // 2 * K, HPAD), f32),
            pltpu.VMEM((SC_RC, K), f32),
            pltpu.VMEM((SC_RC, HID), f32),
            pltpu.VMEM((SC_RC, HID), f32),
            pltpu.SemaphoreType.DMA,
        ],
    )
    mean_p, max_p = run(h, nbr2d, wgt_p)
    return mean_p[:N], max_p[:N]


def _mlp_kernel(x_ref, mean_ref, max_ref,
                Wo1_ref, Wo2_ref, Wo3_ref, bout_ref,
                W2a1_ref, W2a2_ref, b2a_ref, W2b_ref, b2b_ref,
                W2c_ref, b2c_ref, W2d_ref, b2d_ref,
                W3a1_ref, W3a2_ref, W3a3_ref, b3a_ref, W3b_ref, b3b_ref,
                W3c_ref, b3c_ref, W3d_ref, b3d_ref,
                ids_ref, p4_ref):
    x = x_ref[...]
    f32 = jnp.float32
    dot = functools.partial(jnp.dot, preferred_element_type=f32)
    xc = (dot(x, Wo1_ref[...]) + dot(mean_ref[...], Wo2_ref[...])
          + dot(max_ref[...], Wo3_ref[...]) + bout_ref[...])
    x1 = _leaky(xc)
    h2 = _leaky(dot(x, W2a1_ref[...]) + dot(x1, W2a2_ref[...]) + b2a_ref[...])
    h2 = _leaky(dot(h2, W2b_ref[...]) + b2b_ref[...])
    h2 = _leaky(dot(h2, W2c_ref[...]) + b2c_ref[...])
    ids = dot(h2, W2d_ref[...]) + b2d_ref[...]
    h3 = _leaky(dot(x, W3a1_ref[...]) + dot(x1, W3a2_ref[...])
                + dot(ids, W3a3_ref[...]) + b3a_ref[...])
    h3 = _leaky(dot(h3, W3b_ref[...]) + b3b_ref[...])
    h3 = _leaky(dot(h3, W3c_ref[...]) + b3c_ref[...])
    ids_ref[...] = ids
    p4_ref[...] = x[:, ELEM_OFF:ELEM_OFF + OUT_P4] + (dot(h3, W3d_ref[...]) + b3d_ref[...])


def kernel(x, Ws, bs, Wh, bh, Wout, bout,
           W2a, b2a, W2b, b2b, W2c, b2c, W2d, b2d,
           W3a, b3a, W3b, b3b, W3c, b3c, W3d, b3d):
    f32 = jnp.float32
    Wh_p = jnp.pad(Wh, ((0, 0), (0, HPAD - HID)))
    bh_p = jnp.pad(bh, ((0, HPAD - HID),))
    s, sq, h = pl.pallas_call(
        _sh_kernel,
        out_shape=(jax.ShapeDtypeStruct((N, SPACE), f32),
                   jax.ShapeDtypeStruct((N, 1), f32),
                   jax.ShapeDtypeStruct((N, HPAD), f32)),
        interpret=_INTERPRET,
    )(x, Ws, bs, Wh_p, bh_p)

    sT = s.T
    sq_row = sq.T

    grid_b = N // TR
    nbr, wgt = pl.pallas_call(
        _knn_kernel,
        grid=(grid_b,),
        in_specs=[
            pl.BlockSpec((TR, 1), lambda i: (i, 0)),
            pl.BlockSpec((TR, SPACE), lambda i: (i, 0)),
            pl.BlockSpec((SPACE, N), lambda i: (0, 0)),
            pl.BlockSpec((1, N), lambda i: (0, 0)),
        ],
        out_specs=[
            pl.BlockSpec((TR, K), lambda i: (i, 0)),
            pl.BlockSpec((TR, K), lambda i: (i, 0)),
        ],
        out_shape=(jax.ShapeDtypeStruct((N, K), jnp.int32),
                   jax.ShapeDtypeStruct((N, K), f32)),
        scratch_shapes=[pltpu.VMEM((TR, N), f32)],
        interpret=_INTERPRET,
    )(sq, s, sT, sq_row)

    mean_agg, max_agg = _sc_aggregate(h, nbr, wgt)

    dec = IN_DIM + ENC
    grid_c = N // TM
    full = lambda shp: pl.BlockSpec(shp, lambda i: tuple(0 for _ in shp))
    w_ins = [Wout[:IN_DIM], Wout[IN_DIM:IN_DIM + HID], Wout[IN_DIM + HID:], bout,
             W2a[:IN_DIM], W2a[IN_DIM:], b2a, W2b, b2b, W2c, b2c, W2d, b2d,
             W3a[:IN_DIM], W3a[IN_DIM:dec], W3a[dec:], b3a, W3b, b3b,
             W3c, b3c, W3d, b3d]
    cand_ids, cand_p4 = pl.pallas_call(
        _mlp_kernel,
        grid=(grid_c,),
        in_specs=[
            pl.BlockSpec((TM, IN_DIM), lambda i: (i, 0)),
            pl.BlockSpec((TM, HID), lambda i: (i, 0)),
            pl.BlockSpec((TM, HID), lambda i: (i, 0)),
        ] + [full(w.shape) for w in w_ins],
        out_specs=[
            pl.BlockSpec((TM, OUT_ID), lambda i: (i, 0)),
            pl.BlockSpec((TM, OUT_P4), lambda i: (i, 0)),
        ],
        out_shape=(jax.ShapeDtypeStruct((N, OUT_ID), f32),
                   jax.ShapeDtypeStruct((N, OUT_P4), f32)),
        interpret=_INTERPRET,
    )(x, mean_agg, max_agg, *w_ins)

    src = nbr.reshape(-1)
    dst = jnp.repeat(jnp.arange(N, dtype=jnp.int32), K)
    edge_index = jnp.stack([src, dst]).astype(jnp.int32)
    return cand_ids, cand_p4, edge_index


# speedup vs baseline: 1.2961x; 1.2961x over previous
import functools

import jax
import jax.numpy as jnp
from jax import lax
from jax.experimental import pallas as pl
from jax.experimental.pallas import tpu as pltpu
from jax.experimental.pallas import tpu_sc as plsc

N = 10000
IN_DIM = 15
HID = 32
ENC = 256
SPACE = 2
K = 16
OUT_ID = 6
OUT_P4 = 4
ELEM_OFF = 3

TR = 1000
TM = 2000

_INTERPRET = False


def _leaky(v):
    return jnp.where(v >= 0, v, 0.01 * v)


def _sh_kernel(x_ref, Ws_ref, bs_ref, Wh_ref, bh_ref, s_ref, sq_ref, h_ref):
    x = x_ref[...]
    s = jnp.dot(x, Ws_ref[...], preferred_element_type=jnp.float32) + bs_ref[...]
    s_ref[...] = s
    sq_ref[...] = jnp.sum(s * s, axis=1, keepdims=True)
    h_ref[...] = jnp.dot(x, Wh_ref[...], preferred_element_type=jnp.float32) + bh_ref[...]


def _knn_kernel(sq_col_ref, s_tile_ref, sT_ref, sq_row_ref,
                nbr_ref, wgt_ref, d2_ref):
    i = pl.program_id(0)
    p = jnp.dot(s_tile_ref[...], sT_ref[...], preferred_element_type=jnp.float32)
    d2 = (sq_col_ref[...] + sq_row_ref[...]) - 2.0 * p
    rows = jax.lax.broadcasted_iota(jnp.int32, (TR, N), 0) + i * TR
    cols = jax.lax.broadcasted_iota(jnp.int32, (TR, N), 1)
    d2_ref[...] = jnp.where(cols == rows, jnp.inf, d2)

    def body(k, carry):
        nbr_acc, wgt_acc = carry
        d2c = d2_ref[...]
        m = jnp.min(d2c, axis=1, keepdims=True)
        idx = jnp.min(jnp.where(d2c == m, cols, N), axis=1, keepdims=True)
        d2_ref[...] = jnp.where(cols == idx, jnp.inf, d2c)
        lane = jax.lax.broadcasted_iota(jnp.int32, (TR, K), 1)
        nbr_acc = jnp.where(lane == k, idx, nbr_acc)
        wgt_acc = jnp.where(lane == k, jnp.exp(-10.0 * m), wgt_acc)
        return nbr_acc, wgt_acc

    nbr, wgt = jax.lax.fori_loop(
        0, K,
        body,
        (jnp.zeros((TR, K), jnp.int32),
         jnp.zeros((TR, K), jnp.float32)),
    )
    nbr_ref[...] = nbr
    wgt_ref[...] = wgt


N_P = 10240
HPAD = 128
SC_RC = 64
SC_CH = 5
SC_G = SC_RC * K // 128
SC_G2 = SC_G // 2


def _sc_agg_body(h_hbm, nbr2d_hbm, wgt_hbm, mean_hbm, max_hbm,
                 idx_v, rows_v, wv_all, mean_v, max_v, sem):
    wid = lax.axis_index("s") * 2 + lax.axis_index("c")
    k_bcast_idx = [jnp.full((16,), kk, jnp.int32)[:, None] for kk in range(K)]
    dnums = lax.GatherDimensionNumbers(offset_dims=(), collapsed_slice_dims=(0,),
                                       start_index_map=(0,))

    for c in range(SC_CH):
        r0 = pl.multiple_of(wid * (SC_RC * SC_CH) + c * SC_RC, SC_RC)
        pltpu.sync_copy(nbr2d_hbm.at[pl.ds(pl.multiple_of(r0 // 8, 8), SC_G)], idx_v)
        pltpu.sync_copy(wgt_hbm.at[pl.ds(r0, SC_RC)], wv_all)
        for half in range(2):
            copies = [pltpu.async_copy(h_hbm.at[idx_v.at[half * SC_G2 + j]],
                                       rows_v.at[pl.ds(j * 128, 128)], sem)
                      for j in range(SC_G2)]
            for cp in copies:
                cp.wait()

            def row_body(r, _):
                wv = wv_all[half * (SC_RC // 2) + r, :]
                acc0 = jnp.zeros((16,), jnp.float32)
                acc1 = jnp.zeros((16,), jnp.float32)
                mx0 = jnp.full((16,), -jnp.inf, jnp.float32)
                mx1 = jnp.full((16,), -jnp.inf, jnp.float32)
                for kk in range(K):
                    wb = lax.gather(wv, k_bcast_idx[kk], dnums, (1,),
                                    mode=lax.GatherScatterMode.PROMISE_IN_BOUNDS)
                    g0 = rows_v[r * K + kk, 0:16]
                    g1 = rows_v[r * K + kk, 16:32]
                    m0 = wb * g0
                    m1 = wb * g1
                    acc0 = acc0 + m0
                    acc1 = acc1 + m1
                    mx0 = jnp.maximum(mx0, m0)
                    mx1 = jnp.maximum(mx1, m1)
                ro = half * (SC_RC // 2) + r
                mean_v[ro, 0:16] = acc0 * (1.0 / K)
                mean_v[ro, 16:32] = acc1 * (1.0 / K)
                max_v[ro, 0:16] = mx0
                max_v[ro, 16:32] = mx1
                return 0

            lax.fori_loop(0, SC_RC // 2, row_body, 0)
        pltpu.sync_copy(mean_v, mean_hbm.at[pl.ds(r0, SC_RC)])
        pltpu.sync_copy(max_v, max_hbm.at[pl.ds(r0, SC_RC)])


def _sc_aggregate(h, nbr, wgt):
    pad = N_P - N
    nbr2d = jnp.pad(nbr, ((0, pad), (0, 0))).reshape(N_P * K // 128, 128)
    wgt_p = jnp.pad(wgt, ((0, pad), (0, 0)))
    f32 = jnp.float32
    mesh = plsc.VectorSubcoreMesh(core_axis_name="c", subcore_axis_name="s")
    run = pl.kernel(
        _sc_agg_body,
        mesh=mesh,
        out_type=(jax.ShapeDtypeStruct((N_P, HID), f32),
                  jax.ShapeDtypeStruct((N_P, HID), f32)),
        scratch_types=[
            pltpu.VMEM((SC_G, 128), jnp.int32),
            pltpu.VMEM((SC_RC // 2 * K, HPAD), f32),
            pltpu.VMEM((SC_RC, K), f32),
            pltpu.VMEM((SC_RC, HID), f32),
            pltpu.VMEM((SC_RC, HID), f32),
            pltpu.SemaphoreType.DMA,
        ],
    )
    mean_p, max_p = run(h, nbr2d, wgt_p)
    return mean_p[:N], max_p[:N]


def _mlp_kernel(x_ref, mean_ref, max_ref,
                Wo1_ref, Wo2_ref, Wo3_ref, bout_ref,
                W2a1_ref, W2a2_ref, b2a_ref, W2b_ref, b2b_ref,
                W2c_ref, b2c_ref, W2d_ref, b2d_ref,
                W3a1_ref, W3a2_ref, W3a3_ref, b3a_ref, W3b_ref, b3b_ref,
                W3c_ref, b3c_ref, W3d_ref, b3d_ref,
                ids_ref, p4_ref):
    x = x_ref[...]
    f32 = jnp.float32
    dot = functools.partial(jnp.dot, preferred_element_type=f32)
    xc = (dot(x, Wo1_ref[...]) + dot(mean_ref[...], Wo2_ref[...])
          + dot(max_ref[...], Wo3_ref[...]) + bout_ref[...])
    x1 = _leaky(xc)
    h2 = _leaky(dot(x, W2a1_ref[...]) + dot(x1, W2a2_ref[...]) + b2a_ref[...])
    h2 = _leaky(dot(h2, W2b_ref[...]) + b2b_ref[...])
    h2 = _leaky(dot(h2, W2c_ref[...]) + b2c_ref[...])
    ids = dot(h2, W2d_ref[...]) + b2d_ref[...]
    h3 = _leaky(dot(x, W3a1_ref[...]) + dot(x1, W3a2_ref[...])
                + dot(ids, W3a3_ref[...]) + b3a_ref[...])
    h3 = _leaky(dot(h3, W3b_ref[...]) + b3b_ref[...])
    h3 = _leaky(dot(h3, W3c_ref[...]) + b3c_ref[...])
    ids_ref[...] = ids
    p4_ref[...] = x[:, ELEM_OFF:ELEM_OFF + OUT_P4] + (dot(h3, W3d_ref[...]) + b3d_ref[...])


def kernel(x, Ws, bs, Wh, bh, Wout, bout,
           W2a, b2a, W2b, b2b, W2c, b2c, W2d, b2d,
           W3a, b3a, W3b, b3b, W3c, b3c, W3d, b3d):
    f32 = jnp.float32
    Wh_p = jnp.pad(Wh, ((0, 0), (0, HPAD - HID)))
    bh_p = jnp.pad(bh, ((0, HPAD - HID),))
    s, sq, h = pl.pallas_call(
        _sh_kernel,
        out_shape=(jax.ShapeDtypeStruct((N, SPACE), f32),
                   jax.ShapeDtypeStruct((N, 1), f32),
                   jax.ShapeDtypeStruct((N, HPAD), f32)),
        interpret=_INTERPRET,
    )(x, Ws, bs, Wh_p, bh_p)

    sT = s.T
    sq_row = sq.T

    grid_b = N // TR
    nbr, wgt = pl.pallas_call(
        _knn_kernel,
        grid=(grid_b,),
        in_specs=[
            pl.BlockSpec((TR, 1), lambda i: (i, 0)),
            pl.BlockSpec((TR, SPACE), lambda i: (i, 0)),
            pl.BlockSpec((SPACE, N), lambda i: (0, 0)),
            pl.BlockSpec((1, N), lambda i: (0, 0)),
        ],
        out_specs=[
            pl.BlockSpec((TR, K), lambda i: (i, 0)),
            pl.BlockSpec((TR, K), lambda i: (i, 0)),
        ],
        out_shape=(jax.ShapeDtypeStruct((N, K), jnp.int32),
                   jax.ShapeDtypeStruct((N, K), f32)),
        scratch_shapes=[pltpu.VMEM((TR, N), f32)],
        interpret=_INTERPRET,
    )(sq, s, sT, sq_row)

    mean_agg, max_agg = _sc_aggregate(h, nbr, wgt)

    dec = IN_DIM + ENC
    grid_c = N // TM
    full = lambda shp: pl.BlockSpec(shp, lambda i: tuple(0 for _ in shp))
    w_ins = [Wout[:IN_DIM], Wout[IN_DIM:IN_DIM + HID], Wout[IN_DIM + HID:], bout,
             W2a[:IN_DIM], W2a[IN_DIM:], b2a, W2b, b2b, W2c, b2c, W2d, b2d,
             W3a[:IN_DIM], W3a[IN_DIM:dec], W3a[dec:], b3a, W3b, b3b,
             W3c, b3c, W3d, b3d]
    cand_ids, cand_p4 = pl.pallas_call(
        _mlp_kernel,
        grid=(grid_c,),
        in_specs=[
            pl.BlockSpec((TM, IN_DIM), lambda i: (i, 0)),
            pl.BlockSpec((TM, HID), lambda i: (i, 0)),
            pl.BlockSpec((TM, HID), lambda i: (i, 0)),
        ] + [full(w.shape) for w in w_ins],
        out_specs=[
            pl.BlockSpec((TM, OUT_ID), lambda i: (i, 0)),
            pl.BlockSpec((TM, OUT_P4), lambda i: (i, 0)),
        ],
        out_shape=(jax.ShapeDtypeStruct((N, OUT_ID), f32),
                   jax.ShapeDtypeStruct((N, OUT_P4), f32)),
        interpret=_INTERPRET,
    )(x, mean_agg, max_agg, *w_ins)

    src = nbr.reshape(-1)
    dst = jnp.repeat(jnp.arange(N, dtype=jnp.int32), K)
    edge_index = jnp.stack([src, dst]).astype(jnp.int32)
    return cand_ids, cand_p4, edge_index


# speedup vs baseline: 1.2963x; 1.0001x over previous
import functools

import jax
import jax.numpy as jnp
from jax import lax
from jax.experimental import pallas as pl
from jax.experimental.pallas import tpu as pltpu
from jax.experimental.pallas import tpu_sc as plsc

N = 10000
IN_DIM = 15
HID = 32
ENC = 256
SPACE = 2
K = 16
OUT_ID = 6
OUT_P4 = 4
ELEM_OFF = 3

TR = 1000
TM = 2000


def _leaky(v):
    return jnp.where(v >= 0, v, 0.01 * v)


def _sh_kernel(x_ref, Ws_ref, bs_ref, Wh_ref, bh_ref, s_ref, sq_ref, h_ref):
    x = x_ref[...]
    s = jnp.dot(x, Ws_ref[...], preferred_element_type=jnp.float32) + bs_ref[...]
    s_ref[...] = s
    sq_ref[...] = jnp.sum(s * s, axis=1, keepdims=True)
    h_ref[...] = jnp.dot(x, Wh_ref[...], preferred_element_type=jnp.float32) + bh_ref[...]


def _knn_kernel(sq_col_ref, s_tile_ref, sT_ref, sq_row_ref,
                nbr_ref, wgt_ref, d2_ref):
    i = pl.program_id(0)
    p = jnp.dot(s_tile_ref[...], sT_ref[...], preferred_element_type=jnp.float32)
    d2 = (sq_col_ref[...] + sq_row_ref[...]) - 2.0 * p
    rows = jax.lax.broadcasted_iota(jnp.int32, (TR, N), 0) + i * TR
    cols = jax.lax.broadcasted_iota(jnp.int32, (TR, N), 1)
    d2_ref[...] = jnp.where(cols == rows, jnp.inf, d2)

    def body(k, carry):
        nbr_acc, wgt_acc = carry
        d2c = d2_ref[...]
        m = jnp.min(d2c, axis=1, keepdims=True)
        idx = jnp.min(jnp.where(d2c == m, cols, N), axis=1, keepdims=True)
        d2_ref[...] = jnp.where(cols == idx, jnp.inf, d2c)
        lane = jax.lax.broadcasted_iota(jnp.int32, (TR, K), 1)
        nbr_acc = jnp.where(lane == k, idx, nbr_acc)
        wgt_acc = jnp.where(lane == k, jnp.exp(-10.0 * m), wgt_acc)
        return nbr_acc, wgt_acc

    nbr, wgt = jax.lax.fori_loop(
        0, K,
        body,
        (jnp.zeros((TR, K), jnp.int32),
         jnp.zeros((TR, K), jnp.float32)),
    )
    nbr_ref[...] = nbr
    wgt_ref[...] = wgt


N_P = 10240
HPAD = 128
SC_RC = 64
SC_CH = 5
SC_G = SC_RC * K // 128
SC_G2 = SC_G // 2


def _sc_agg_body(h_hbm, nbr2d_hbm, wgt_hbm, mean_hbm, max_hbm,
                 idx_v, rows_v, wv_all, mean_v, max_v, sem):
    wid = lax.axis_index("s") * 2 + lax.axis_index("c")
    k_bcast_idx = [jnp.full((16,), kk, jnp.int32)[:, None] for kk in range(K)]
    dnums = lax.GatherDimensionNumbers(offset_dims=(), collapsed_slice_dims=(0,),
                                       start_index_map=(0,))

    for c in range(SC_CH):
        r0 = pl.multiple_of(wid * (SC_RC * SC_CH) + c * SC_RC, SC_RC)
        pltpu.sync_copy(nbr2d_hbm.at[pl.ds(pl.multiple_of(r0 // 8, 8), SC_G)], idx_v)
        pltpu.sync_copy(wgt_hbm.at[pl.ds(r0, SC_RC)], wv_all)
        for half in range(2):
            copies = [pltpu.async_copy(h_hbm.at[idx_v.at[half * SC_G2 + j]],
                                       rows_v.at[pl.ds(j * 128, 128)], sem)
                      for j in range(SC_G2)]
            for cp in copies:
                cp.wait()

            def row_body(r, _):
                wv = wv_all[half * (SC_RC // 2) + r, :]
                acc0 = jnp.zeros((16,), jnp.float32)
                acc1 = jnp.zeros((16,), jnp.float32)
                mx0 = jnp.full((16,), -jnp.inf, jnp.float32)
                mx1 = jnp.full((16,), -jnp.inf, jnp.float32)
                for kk in range(K):
                    wb = lax.gather(wv, k_bcast_idx[kk], dnums, (1,),
                                    mode=lax.GatherScatterMode.PROMISE_IN_BOUNDS)
                    g0 = rows_v[r * K + kk, 0:16]
                    g1 = rows_v[r * K + kk, 16:32]
                    m0 = wb * g0
                    m1 = wb * g1
                    acc0 = acc0 + m0
                    acc1 = acc1 + m1
                    mx0 = jnp.maximum(mx0, m0)
                    mx1 = jnp.maximum(mx1, m1)
                ro = half * (SC_RC // 2) + r
                mean_v[ro, 0:16] = acc0 * (1.0 / K)
                mean_v[ro, 16:32] = acc1 * (1.0 / K)
                max_v[ro, 0:16] = mx0
                max_v[ro, 16:32] = mx1
                return 0

            lax.fori_loop(0, SC_RC // 2, row_body, 0)
        pltpu.sync_copy(mean_v, mean_hbm.at[pl.ds(r0, SC_RC)])
        pltpu.sync_copy(max_v, max_hbm.at[pl.ds(r0, SC_RC)])


def _sc_aggregate(h, nbr, wgt):
    pad = N_P - N
    nbr2d = jnp.pad(nbr, ((0, pad), (0, 0))).reshape(N_P * K // 128, 128)
    wgt_p = jnp.pad(wgt, ((0, pad), (0, 0)))
    f32 = jnp.float32
    mesh = plsc.VectorSubcoreMesh(core_axis_name="c", subcore_axis_name="s")
    run = pl.kernel(
        _sc_agg_body,
        mesh=mesh,
        out_type=(jax.ShapeDtypeStruct((N_P, HID), f32),
                  jax.ShapeDtypeStruct((N_P, HID), f32)),
        scratch_types=[
            pltpu.VMEM((SC_G, 128), jnp.int32),
            pltpu.VMEM((SC_RC // 2 * K, HPAD), f32),
            pltpu.VMEM((SC_RC, K), f32),
            pltpu.VMEM((SC_RC, HID), f32),
            pltpu.VMEM((SC_RC, HID), f32),
            pltpu.SemaphoreType.DMA,
        ],
    )
    mean_p, max_p = run(h, nbr2d, wgt_p)
    return mean_p[:N], max_p[:N]


def _mlp_kernel(x_ref, mean_ref, max_ref,
                Wo1_ref, Wo2_ref, Wo3_ref, bout_ref,
                W2a1_ref, W2a2_ref, b2a_ref, W2b_ref, b2b_ref,
                W2c_ref, b2c_ref, W2d_ref, b2d_ref,
                W3a1_ref, W3a2_ref, W3a3_ref, b3a_ref, W3b_ref, b3b_ref,
                W3c_ref, b3c_ref, W3d_ref, b3d_ref,
                ids_ref, p4_ref):
    x = x_ref[...]
    f32 = jnp.float32
    dot = functools.partial(jnp.dot, preferred_element_type=f32)
    xc = (dot(x, Wo1_ref[...]) + dot(mean_ref[...], Wo2_ref[...])
          + dot(max_ref[...], Wo3_ref[...]) + bout_ref[...])
    x1 = _leaky(xc)
    h2 = _leaky(dot(x, W2a1_ref[...]) + dot(x1, W2a2_ref[...]) + b2a_ref[...])
    h2 = _leaky(dot(h2, W2b_ref[...]) + b2b_ref[...])
    h2 = _leaky(dot(h2, W2c_ref[...]) + b2c_ref[...])
    ids = dot(h2, W2d_ref[...]) + b2d_ref[...]
    h3 = _leaky(dot(x, W3a1_ref[...]) + dot(x1, W3a2_ref[...])
                + dot(ids, W3a3_ref[...]) + b3a_ref[...])
    h3 = _leaky(dot(h3, W3b_ref[...]) + b3b_ref[...])
    h3 = _leaky(dot(h3, W3c_ref[...]) + b3c_ref[...])
    ids_ref[...] = ids
    p4_ref[...] = x[:, ELEM_OFF:ELEM_OFF + OUT_P4] + (dot(h3, W3d_ref[...]) + b3d_ref[...])


def kernel(x, Ws, bs, Wh, bh, Wout, bout,
           W2a, b2a, W2b, b2b, W2c, b2c, W2d, b2d,
           W3a, b3a, W3b, b3b, W3c, b3c, W3d, b3d):
    f32 = jnp.float32
    Wh_p = jnp.pad(Wh, ((0, 0), (0, HPAD - HID)))
    bh_p = jnp.pad(bh, ((0, HPAD - HID),))
    s, sq, h = pl.pallas_call(
        _sh_kernel,
        out_shape=(jax.ShapeDtypeStruct((N, SPACE), f32),
                   jax.ShapeDtypeStruct((N, 1), f32),
                   jax.ShapeDtypeStruct((N, HPAD), f32)),
    )(x, Ws, bs, Wh_p, bh_p)

    sT = s.T
    sq_row = sq.T

    grid_b = N // TR
    nbr, wgt = pl.pallas_call(
        _knn_kernel,
        grid=(grid_b,),
        in_specs=[
            pl.BlockSpec((TR, 1), lambda i: (i, 0)),
            pl.BlockSpec((TR, SPACE), lambda i: (i, 0)),
            pl.BlockSpec((SPACE, N), lambda i: (0, 0)),
            pl.BlockSpec((1, N), lambda i: (0, 0)),
        ],
        out_specs=[
            pl.BlockSpec((TR, K), lambda i: (i, 0)),
            pl.BlockSpec((TR, K), lambda i: (i, 0)),
        ],
        out_shape=(jax.ShapeDtypeStruct((N, K), jnp.int32),
                   jax.ShapeDtypeStruct((N, K), f32)),
        scratch_shapes=[pltpu.VMEM((TR, N), f32)],
    )(sq, s, sT, sq_row)

    mean_agg, max_agg = _sc_aggregate(h, nbr, wgt)

    dec = IN_DIM + ENC
    grid_c = N // TM
    full = lambda shp: pl.BlockSpec(shp, lambda i: tuple(0 for _ in shp))
    w_ins = [Wout[:IN_DIM], Wout[IN_DIM:IN_DIM + HID], Wout[IN_DIM + HID:], bout,
             W2a[:IN_DIM], W2a[IN_DIM:], b2a, W2b, b2b, W2c, b2c, W2d, b2d,
             W3a[:IN_DIM], W3a[IN_DIM:dec], W3a[dec:], b3a, W3b, b3b,
             W3c, b3c, W3d, b3d]
    cand_ids, cand_p4 = pl.pallas_call(
        _mlp_kernel,
        grid=(grid_c,),
        in_specs=[
            pl.BlockSpec((TM, IN_DIM), lambda i: (i, 0)),
            pl.BlockSpec((TM, HID), lambda i: (i, 0)),
            pl.BlockSpec((TM, HID), lambda i: (i, 0)),
        ] + [full(w.shape) for w in w_ins],
        out_specs=[
            pl.BlockSpec((TM, OUT_ID), lambda i: (i, 0)),
            pl.BlockSpec((TM, OUT_P4), lambda i: (i, 0)),
        ],
        out_shape=(jax.ShapeDtypeStruct((N, OUT_ID), f32),
                   jax.ShapeDtypeStruct((N, OUT_P4), f32)),
    )(x, mean_agg, max_agg, *w_ins)

    src = nbr.reshape(-1)
    dst = jnp.repeat(jnp.arange(N, dtype=jnp.int32), K)
    edge_index = jnp.stack([src, dst]).astype(jnp.int32)
    return cand_ids, cand_p4, edge_index
